# baseline (speedup 1.0000x reference)
import functools

import jax
import jax.numpy as jnp
from jax.experimental import pallas as pl
from jax.experimental.pallas import tpu as pltpu

EPS = 1e-6
_VMEM_LIMIT = 96 * 1024 * 1024


def _col(v):
    return v.reshape(-1, 1)


def _r2(v):
    return v.reshape(1, -1)


def _ln_cols(y, lnw_col, lnb_col):
    mean = jnp.mean(y, axis=0, keepdims=True)
    yc = y - mean
    var = jnp.mean(yc * yc, axis=0, keepdims=True)
    return yc * jax.lax.rsqrt(var + EPS) * lnw_col + lnb_col


def _dw7x7_t(x, dwt, pad_ref, *, H, W):
    C, M = x.shape
    Wp = W + 6
    Mp = H * Wp
    P0 = 3 * Wp + 3
    pad_ref[...] = jnp.zeros_like(pad_ref)
    xb = x.astype(jnp.bfloat16)
    for r in range(H):
        pad_ref[:, P0 + r * Wp:P0 + r * Wp + W] = xb[:, r * W:(r + 1) * W]
    acc = jnp.zeros((C, Mp), jnp.float32)
    for kh in range(7):
        for kw in range(7):
            start = kh * Wp + kw
            acc = acc + (pad_ref[:, start:start + Mp]
                         * dwt[:, kh * 7 + kw:kh * 7 + kw + 1])
    return acc


def _compact_w(yp, *, H, W):
    Wp = W + 6
    parts = [yp[:, r * Wp:r * Wp + W] for r in range(H)]
    return jnp.concatenate(parts, axis=1)


def _block_t(x, dww_ref, dwb_col, lnw_col, lnb_col, w1t_ref, b1_col,
             w2t_ref, b2_col, g_col, pad_ref, *, H, W):
    acc = _dw7x7_t(x, dww_ref[...], pad_ref, H=H, W=W)
    h = _compact_w(acc, H=H, W=W) + dwb_col
    y = _ln_cols(h, lnw_col, lnb_col).astype(jnp.bfloat16)
    h1 = jnp.dot(w1t_ref[...], y, preferred_element_type=jnp.float32) + b1_col
    h1 = jax.nn.gelu(h1, approximate=True)
    z = jnp.dot(w2t_ref[...], h1.astype(jnp.bfloat16),
                preferred_element_type=jnp.float32) + b2_col
    return x + g_col * z


def _k1(xp_ref, sw_ref, sb_ref, slnw_ref, slnb_ref,
        dww_ref, dwb_ref, lnw_ref, lnb_ref, w1t_ref, b1_ref, w2t_ref, b2_ref,
        g_ref, dlnw_ref, dlnb_ref, o_ref, pad_ref, *, H, W):
    s = jnp.dot(sw_ref[...], xp_ref[...],
                preferred_element_type=jnp.float32) + _col(sb_ref[...])
    s = _ln_cols(s, _col(slnw_ref[...]), _col(slnb_ref[...]))
    o = _block_t(s, dww_ref, _col(dwb_ref[...]), _col(lnw_ref[...]),
                 _col(lnb_ref[...]), w1t_ref, _col(b1_ref[...]), w2t_ref,
                 _col(b2_ref[...]), _col(g_ref[...]), pad_ref, H=H, W=W)
    y = _ln_cols(o, _col(dlnw_ref[...]), _col(dlnb_ref[...]))
    o_ref[...] = y.astype(o_ref.dtype)


def _k23(p0_ref, p1_ref, p2_ref, p3_ref, dsw_ref, dsb_ref,
         dww_ref, dwb_ref, lnw_ref, lnb_ref, w1t_ref, b1_ref, w2t_ref, b2_ref,
         g_ref, dlnw_ref, dlnb_ref, o_ref, pad_ref, *, H, W, last):
    acc = jnp.dot(dsw_ref[0], p0_ref[...], preferred_element_type=jnp.float32)
    acc = acc + jnp.dot(dsw_ref[1], p1_ref[...],
                        preferred_element_type=jnp.float32)
    acc = acc + jnp.dot(dsw_ref[2], p2_ref[...],
                        preferred_element_type=jnp.float32)
    acc = acc + jnp.dot(dsw_ref[3], p3_ref[...],
                        preferred_element_type=jnp.float32)
    x = acc + _col(dsb_ref[...])
    o = _block_t(x, dww_ref, _col(dwb_ref[...]), _col(lnw_ref[...]),
                 _col(lnb_ref[...]), w1t_ref, _col(b1_ref[...]), w2t_ref,
                 _col(b2_ref[...]), _col(g_ref[...]), pad_ref, H=H, W=W)
    if last:
        o_ref[...] = o
    else:
        y = _ln_cols(o, _col(dlnw_ref[...]), _col(dlnb_ref[...]))
        o_ref[...] = y.astype(o_ref.dtype)


def _full_spec(shape):
    n = len(shape)
    return pl.BlockSpec(shape, lambda i: (0,) * n)


def _img_spec(C, M):
    return pl.BlockSpec((None, C, M), lambda n: (n, 0, 0))


def _cp():
    return pltpu.CompilerParams(dimension_semantics=("parallel",),
                                vmem_limit_bytes=_VMEM_LIMIT)


def _extract4(y, H, W):
    N, C, _ = y.shape
    y4 = y.reshape(N, C, H, W)
    M1 = (H // 2) * (W // 2)
    return [y4[:, :, dh::2, dw::2].reshape(N, C, M1)
            for dh in range(2) for dw in range(2)]


def kernel(images, stem_conv_w, stem_conv_b, stem_ln_w, stem_ln_b, blk0_dw_w, blk0_dw_b, blk0_ln_w, blk0_ln_b, blk0_w1, blk0_b1, blk0_w2, blk0_b2, blk0_gamma, ds1_ln_w, ds1_ln_b, ds1_conv_w, ds1_conv_b, blk1_dw_w, blk1_dw_b, blk1_ln_w, blk1_ln_b, blk1_w1, blk1_b1, blk1_w2, blk1_b2, blk1_gamma, ds2_ln_w, ds2_ln_b, ds2_conv_w, ds2_conv_b, blk2_dw_w, blk2_dw_b, blk2_ln_w, blk2_ln_b, blk2_w1, blk2_b1, blk2_w2, blk2_b2, blk2_gamma, ds3_ln_w, ds3_ln_b, ds3_conv_w, ds3_conv_b, blk3_dw_w, blk3_dw_b, blk3_ln_w, blk3_ln_b, blk3_w1, blk3_b1, blk3_w2, blk3_b2, blk3_gamma):
    N, Cin, Him, Wim = images.shape
    ps = 4
    H0, W0 = Him // ps, Wim // ps
    M0 = H0 * W0
    C0, C1, C2 = blk0_dw_w.shape[1], blk1_dw_w.shape[1], blk2_dw_w.shape[1]
    K0 = Cin * ps * ps

    xp = images.reshape(N, Cin, H0, ps, W0, ps)
    xp = xp.transpose(0, 1, 3, 5, 2, 4).reshape(N, K0, M0)
    xp = xp.astype(jnp.bfloat16)
    swt = stem_conv_w.reshape(ps, ps, Cin, C0).transpose(3, 2, 0, 1)
    swt = swt.reshape(C0, K0).astype(jnp.bfloat16)

    Wp0 = W0 + 6
    k1 = functools.partial(_k1, H=H0, W=W0)
    y1 = pl.pallas_call(
        k1,
        out_shape=jax.ShapeDtypeStruct((N, C0, M0), jnp.bfloat16),
        grid=(N,),
        in_specs=[
            _img_spec(K0, M0),
            _full_spec((C0, K0)),
            _full_spec((1, C0)), _full_spec((1, C0)), _full_spec((1, C0)),
            _full_spec((C0, 49)), _full_spec((1, C0)),
            _full_spec((1, C0)), _full_spec((1, C0)),
            _full_spec((4 * C0, C0)), _full_spec((1, 4 * C0)),
            _full_spec((C0, 4 * C0)), _full_spec((1, C0)),
            _full_spec((1, C0)),
            _full_spec((1, C0)), _full_spec((1, C0)),
        ],
        out_specs=_img_spec(C0, M0),
        scratch_shapes=[pltpu.VMEM((C0, (H0 + 6) * Wp0 + 8), jnp.bfloat16)],
        compiler_params=_cp(),
    )(xp, swt, _r2(stem_conv_b), _r2(stem_ln_w), _r2(stem_ln_b),
      blk0_dw_w.T, _r2(blk0_dw_b), _r2(blk0_ln_w), _r2(blk0_ln_b),
      blk0_w1.T.astype(jnp.bfloat16), _r2(blk0_b1),
      blk0_w2.T.astype(jnp.bfloat16), _r2(blk0_b2), _r2(blk0_gamma),
      _r2(ds1_ln_w), _r2(ds1_ln_b))

    H1, W1 = H0 // 2, W0 // 2
    M1 = H1 * W1
    p = _extract4(y1, H0, W0)
    ds1wt = ds1_conv_w.transpose(0, 2, 1).astype(jnp.bfloat16)
    Wp1 = W1 + 6
    k2 = functools.partial(_k23, H=H1, W=W1, last=False)
    y2 = pl.pallas_call(
        k2,
        out_shape=jax.ShapeDtypeStruct((N, C1, M1), jnp.bfloat16),
        grid=(N,),
        in_specs=[
            _img_spec(C0, M1), _img_spec(C0, M1),
            _img_spec(C0, M1), _img_spec(C0, M1),
            _full_spec((4, C1, C0)), _full_spec((1, C1)),
            _full_spec((C1, 49)), _full_spec((1, C1)),
            _full_spec((1, C1)), _full_spec((1, C1)),
            _full_spec((4 * C1, C1)), _full_spec((1, 4 * C1)),
            _full_spec((C1, 4 * C1)), _full_spec((1, C1)),
            _full_spec((1, C1)),
            _full_spec((1, C1)), _full_spec((1, C1)),
        ],
        out_specs=_img_spec(C1, M1),
        scratch_shapes=[pltpu.VMEM((C1, (H1 + 6) * Wp1 + 8), jnp.bfloat16)],
        compiler_params=_cp(),
    )(*p, ds1wt, _r2(ds1_conv_b),
      blk1_dw_w.T, _r2(blk1_dw_b), _r2(blk1_ln_w), _r2(blk1_ln_b),
      blk1_w1.T.astype(jnp.bfloat16), _r2(blk1_b1),
      blk1_w2.T.astype(jnp.bfloat16), _r2(blk1_b2), _r2(blk1_gamma),
      _r2(ds2_ln_w), _r2(ds2_ln_b))

    H2, W2 = H1 // 2, W1 // 2
    M2 = H2 * W2
    p = _extract4(y2, H1, W1)
    ds2wt = ds2_conv_w.transpose(0, 2, 1).astype(jnp.bfloat16)
    Wp2 = W2 + 6
    k3 = functools.partial(_k23, H=H2, W=W2, last=True)
    out = pl.pallas_call(
        k3,
        out_shape=jax.ShapeDtypeStruct((N, C2, M2), jnp.float32),
        grid=(N,),
        in_specs=[
            _img_spec(C1, M2), _img_spec(C1, M2),
            _img_spec(C1, M2), _img_spec(C1, M2),
            _full_spec((4, C2, C1)), _full_spec((1, C2)),
            _full_spec((C2, 49)), _full_spec((1, C2)),
            _full_spec((1, C2)), _full_spec((1, C2)),
            _full_spec((4 * C2, C2)), _full_spec((1, 4 * C2)),
            _full_spec((C2, 4 * C2)), _full_spec((1, C2)),
            _full_spec((1, C2)),
            _full_spec((1, C2)), _full_spec((1, C2)),
        ],
        out_specs=_img_spec(C2, M2),
        scratch_shapes=[pltpu.VMEM((C2, (H2 + 6) * Wp2 + 8), jnp.bfloat16)],
        compiler_params=_cp(),
    )(*p, ds2wt, _r2(ds2_conv_b),
      blk2_dw_w.T, _r2(blk2_dw_b), _r2(blk2_ln_w), _r2(blk2_ln_b),
      blk2_w1.T.astype(jnp.bfloat16), _r2(blk2_b1),
      blk2_w2.T.astype(jnp.bfloat16), _r2(blk2_b2), _r2(blk2_gamma),
      _r2(ds3_ln_w), _r2(ds3_ln_b))

    return out.reshape(N, C2, H2, W2)


# speedup vs baseline: 2.4589x; 2.4589x over previous
import functools

import jax
import jax.numpy as jnp
from jax.experimental import pallas as pl
from jax.experimental.pallas import tpu as pltpu

EPS = 1e-6
_VMEM_LIMIT = 96 * 1024 * 1024


def _col(v):
    return v.reshape(-1, 1)


def _r2(v):
    return v.reshape(1, -1)


def _ln_cols(y, lnw_col, lnb_col):
    mean = jnp.mean(y, axis=0, keepdims=True)
    yc = y - mean
    var = jnp.mean(yc * yc, axis=0, keepdims=True)
    return yc * jax.lax.rsqrt(var + EPS) * lnw_col + lnb_col


def _dw7x7_t(x, dwt, pad_ref, *, H, W):
    C, M = x.shape
    Wp = W + 6
    Mp = H * Wp
    P0 = 3 * Wp + 3
    pad_ref[...] = jnp.zeros_like(pad_ref)
    xb = x.astype(jnp.bfloat16)
    for r in range(H):
        pad_ref[:, P0 + r * Wp:P0 + r * Wp + W] = xb[:, r * W:(r + 1) * W]
    acc = jnp.zeros((C, Mp), jnp.float32)
    for kh in range(7):
        for kw in range(7):
            start = kh * Wp + kw
            acc = acc + (pad_ref[:, start:start + Mp]
                         * dwt[:, kh * 7 + kw:kh * 7 + kw + 1])
    return acc


def _compact_w(yp, *, H, W):
    Wp = W + 6
    parts = [yp[:, r * Wp:r * Wp + W] for r in range(H)]
    return jnp.concatenate(parts, axis=1)


def _block_t(x, dww_ref, dwb_col, lnw_col, lnb_col, w1t_ref, b1_col,
             w2t_ref, b2_col, g_col, pad_ref, *, H, W):
    acc = _dw7x7_t(x, dww_ref[...], pad_ref, H=H, W=W)
    h = _compact_w(acc, H=H, W=W) + dwb_col
    y = _ln_cols(h, lnw_col, lnb_col).astype(jnp.bfloat16)
    h1 = jnp.dot(w1t_ref[...], y, preferred_element_type=jnp.float32) + b1_col
    h1 = jax.nn.gelu(h1, approximate=True)
    z = jnp.dot(w2t_ref[...], h1.astype(jnp.bfloat16),
                preferred_element_type=jnp.float32) + b2_col
    return x + g_col * z


def _split4(o, dlnw_ref, dlnb_ref, o_refs, t_ref, *, H, W):
    C = o.shape[0]
    y = _ln_cols(o, _col(dlnw_ref[...]), _col(dlnb_ref[...]))
    yt = jnp.transpose(y)
    t_ref[...] = yt.reshape(H, W, C)
    M1 = (H // 2) * (W // 2)
    for k, (dh, dw) in enumerate(((0, 0), (0, 1), (1, 0), (1, 1))):
        o_refs[k][...] = (t_ref[dh::2, dw::2, :].reshape(M1, C)
                          .astype(o_refs[k].dtype))


def _k1(xp_ref, sw_ref, sb_ref, slnw_ref, slnb_ref,
        dww_ref, dwb_ref, lnw_ref, lnb_ref, w1t_ref, b1_ref, w2t_ref, b2_ref,
        g_ref, dlnw_ref, dlnb_ref, o0_ref, o1_ref, o2_ref, o3_ref, pad_ref,
        t_ref, *, H, W):
    s = jnp.dot(sw_ref[...], xp_ref[...],
                preferred_element_type=jnp.float32) + _col(sb_ref[...])
    s = _ln_cols(s, _col(slnw_ref[...]), _col(slnb_ref[...]))
    o = _block_t(s, dww_ref, _col(dwb_ref[...]), _col(lnw_ref[...]),
                 _col(lnb_ref[...]), w1t_ref, _col(b1_ref[...]), w2t_ref,
                 _col(b2_ref[...]), _col(g_ref[...]), pad_ref, H=H, W=W)
    _split4(o, dlnw_ref, dlnb_ref, (o0_ref, o1_ref, o2_ref, o3_ref), t_ref,
            H=H, W=W)


def _k23(p0_ref, p1_ref, p2_ref, p3_ref, dsw_ref, dsb_ref,
         dww_ref, dwb_ref, lnw_ref, lnb_ref, w1t_ref, b1_ref, w2t_ref, b2_ref,
         g_ref, dlnw_ref, dlnb_ref, *refs, H, W, last):
    acc = jnp.dot(p0_ref[...], dsw_ref[0], preferred_element_type=jnp.float32)
    acc = acc + jnp.dot(p1_ref[...], dsw_ref[1],
                        preferred_element_type=jnp.float32)
    acc = acc + jnp.dot(p2_ref[...], dsw_ref[2],
                        preferred_element_type=jnp.float32)
    acc = acc + jnp.dot(p3_ref[...], dsw_ref[3],
                        preferred_element_type=jnp.float32)
    x = jnp.transpose(acc + dsb_ref[...])
    pad_ref = refs[-1] if last else refs[-2]
    o = _block_t(x, dww_ref, _col(dwb_ref[...]), _col(lnw_ref[...]),
                 _col(lnb_ref[...]), w1t_ref, _col(b1_ref[...]), w2t_ref,
                 _col(b2_ref[...]), _col(g_ref[...]), pad_ref, H=H, W=W)
    if last:
        refs[0][...] = o
    else:
        _split4(o, dlnw_ref, dlnb_ref, refs[0:4], refs[-1], H=H, W=W)


def _full_spec(shape):
    n = len(shape)
    return pl.BlockSpec(shape, lambda i: (0,) * n)


def _img_spec(a, b):
    return pl.BlockSpec((None, a, b), lambda n: (n, 0, 0))


def _cp():
    return pltpu.CompilerParams(dimension_semantics=("parallel",),
                                vmem_limit_bytes=_VMEM_LIMIT)


def kernel(images, stem_conv_w, stem_conv_b, stem_ln_w, stem_ln_b, blk0_dw_w, blk0_dw_b, blk0_ln_w, blk0_ln_b, blk0_w1, blk0_b1, blk0_w2, blk0_b2, blk0_gamma, ds1_ln_w, ds1_ln_b, ds1_conv_w, ds1_conv_b, blk1_dw_w, blk1_dw_b, blk1_ln_w, blk1_ln_b, blk1_w1, blk1_b1, blk1_w2, blk1_b2, blk1_gamma, ds2_ln_w, ds2_ln_b, ds2_conv_w, ds2_conv_b, blk2_dw_w, blk2_dw_b, blk2_ln_w, blk2_ln_b, blk2_w1, blk2_b1, blk2_w2, blk2_b2, blk2_gamma, ds3_ln_w, ds3_ln_b, ds3_conv_w, ds3_conv_b, blk3_dw_w, blk3_dw_b, blk3_ln_w, blk3_ln_b, blk3_w1, blk3_b1, blk3_w2, blk3_b2, blk3_gamma):
    N, Cin, Him, Wim = images.shape
    ps = 4
    H0, W0 = Him // ps, Wim // ps
    M0 = H0 * W0
    C0, C1, C2 = blk0_dw_w.shape[1], blk1_dw_w.shape[1], blk2_dw_w.shape[1]
    K0 = Cin * ps * ps

    xp = images.reshape(N, Cin, H0, ps, W0, ps)
    xp = xp.transpose(0, 1, 3, 5, 2, 4).reshape(N, K0, M0)
    xp = xp.astype(jnp.bfloat16)
    swt = stem_conv_w.reshape(ps, ps, Cin, C0).transpose(3, 2, 0, 1)
    swt = swt.reshape(C0, K0).astype(jnp.bfloat16)

    H1, W1 = H0 // 2, W0 // 2
    M1 = H1 * W1
    H2, W2 = H1 // 2, W1 // 2
    M2 = H2 * W2

    Wp0 = W0 + 6
    k1 = functools.partial(_k1, H=H0, W=W0)
    pos_shape1 = jax.ShapeDtypeStruct((N, M1, C0), jnp.bfloat16)
    p = pl.pallas_call(
        k1,
        out_shape=[pos_shape1] * 4,
        grid=(N,),
        in_specs=[
            _img_spec(K0, M0),
            _full_spec((C0, K0)),
            _full_spec((1, C0)), _full_spec((1, C0)), _full_spec((1, C0)),
            _full_spec((C0, 49)), _full_spec((1, C0)),
            _full_spec((1, C0)), _full_spec((1, C0)),
            _full_spec((4 * C0, C0)), _full_spec((1, 4 * C0)),
            _full_spec((C0, 4 * C0)), _full_spec((1, C0)),
            _full_spec((1, C0)),
            _full_spec((1, C0)), _full_spec((1, C0)),
        ],
        out_specs=[_img_spec(M1, C0)] * 4,
        scratch_shapes=[pltpu.VMEM((C0, (H0 + 6) * Wp0 + 8), jnp.bfloat16),
                        pltpu.VMEM((H0, W0, C0), jnp.float32)],
        compiler_params=_cp(),
    )(xp, swt, _r2(stem_conv_b), _r2(stem_ln_w), _r2(stem_ln_b),
      blk0_dw_w.T, _r2(blk0_dw_b), _r2(blk0_ln_w), _r2(blk0_ln_b),
      blk0_w1.T.astype(jnp.bfloat16), _r2(blk0_b1),
      blk0_w2.T.astype(jnp.bfloat16), _r2(blk0_b2), _r2(blk0_gamma),
      _r2(ds1_ln_w), _r2(ds1_ln_b))

    Wp1 = W1 + 6
    k2 = functools.partial(_k23, H=H1, W=W1, last=False)
    pos_shape2 = jax.ShapeDtypeStruct((N, M2, C1), jnp.bfloat16)
    p = pl.pallas_call(
        k2,
        out_shape=[pos_shape2] * 4,
        grid=(N,),
        in_specs=[
            _img_spec(M1, C0), _img_spec(M1, C0),
            _img_spec(M1, C0), _img_spec(M1, C0),
            _full_spec((4, C0, C1)), _full_spec((1, C1)),
            _full_spec((C1, 49)), _full_spec((1, C1)),
            _full_spec((1, C1)), _full_spec((1, C1)),
            _full_spec((4 * C1, C1)), _full_spec((1, 4 * C1)),
            _full_spec((C1, 4 * C1)), _full_spec((1, C1)),
            _full_spec((1, C1)),
            _full_spec((1, C1)), _full_spec((1, C1)),
        ],
        out_specs=[_img_spec(M2, C1)] * 4,
        scratch_shapes=[pltpu.VMEM((C1, (H1 + 6) * Wp1 + 8), jnp.bfloat16),
                        pltpu.VMEM((H1, W1, C1), jnp.float32)],
        compiler_params=_cp(),
    )(*p, ds1_conv_w, _r2(ds1_conv_b),
      blk1_dw_w.T, _r2(blk1_dw_b), _r2(blk1_ln_w), _r2(blk1_ln_b),
      blk1_w1.T.astype(jnp.bfloat16), _r2(blk1_b1),
      blk1_w2.T.astype(jnp.bfloat16), _r2(blk1_b2), _r2(blk1_gamma),
      _r2(ds2_ln_w), _r2(ds2_ln_b))

    Wp2 = W2 + 6
    k3 = functools.partial(_k23, H=H2, W=W2, last=True)
    out = pl.pallas_call(
        k3,
        out_shape=jax.ShapeDtypeStruct((N, C2, M2), jnp.float32),
        grid=(N,),
        in_specs=[
            _img_spec(M2, C1), _img_spec(M2, C1),
            _img_spec(M2, C1), _img_spec(M2, C1),
            _full_spec((4, C1, C2)), _full_spec((1, C2)),
            _full_spec((C2, 49)), _full_spec((1, C2)),
            _full_spec((1, C2)), _full_spec((1, C2)),
            _full_spec((4 * C2, C2)), _full_spec((1, 4 * C2)),
            _full_spec((C2, 4 * C2)), _full_spec((1, C2)),
            _full_spec((1, C2)),
            _full_spec((1, C2)), _full_spec((1, C2)),
        ],
        out_specs=_img_spec(C2, M2),
        scratch_shapes=[pltpu.VMEM((C2, (H2 + 6) * Wp2 + 8), jnp.bfloat16)],
        compiler_params=_cp(),
    )(*p, ds2_conv_w, _r2(ds2_conv_b),
      blk2_dw_w.T, _r2(blk2_dw_b), _r2(blk2_ln_w), _r2(blk2_ln_b),
      blk2_w1.T.astype(jnp.bfloat16), _r2(blk2_b1),
      blk2_w2.T.astype(jnp.bfloat16), _r2(blk2_b2), _r2(blk2_gamma),
      _r2(ds3_ln_w), _r2(ds3_ln_b))

    return out.reshape(N, C2, H2, W2)


# speedup vs baseline: 2.5586x; 1.0405x over previous
import functools

import jax
import jax.numpy as jnp
from jax.experimental import pallas as pl
from jax.experimental.pallas import tpu as pltpu

EPS = 1e-6
_VMEM_LIMIT = 96 * 1024 * 1024


def _col(v):
    return v.reshape(-1, 1)


def _r2(v):
    return v.reshape(1, -1)


def _ln_cols(y, lnw_col, lnb_col):
    mean = jnp.mean(y, axis=0, keepdims=True)
    yc = y - mean
    var = jnp.mean(yc * yc, axis=0, keepdims=True)
    return yc * jax.lax.rsqrt(var + EPS) * lnw_col + lnb_col


def _dw7x7_t(x, dwt, pad_ref, *, H, W):
    C, M = x.shape
    Wp = W + 6
    Mp = H * Wp
    P0 = 3 * Wp + 3
    pad_ref[...] = jnp.zeros_like(pad_ref)
    xb = x.astype(jnp.bfloat16)
    for r in range(H):
        pad_ref[:, P0 + r * Wp:P0 + r * Wp + W] = xb[:, r * W:(r + 1) * W]
    acc = jnp.zeros((C, Mp), jnp.float32)
    for kh in range(7):
        row = jnp.zeros((C, Mp), jnp.bfloat16)
        for kw in range(7):
            start = kh * Wp + kw
            row = row + (pad_ref[:, start:start + Mp]
                         * dwt[:, kh * 7 + kw:kh * 7 + kw + 1])
        acc = acc + row
    return acc


def _compact_w(yp, *, H, W):
    Wp = W + 6
    parts = [yp[:, r * Wp:r * Wp + W] for r in range(H)]
    return jnp.concatenate(parts, axis=1)


def _block_t(x, dww_ref, dwb_col, lnw_col, lnb_col, w1t_ref, b1_col,
             w2t_ref, b2_col, g_col, pad_ref, *, H, W):
    acc = _dw7x7_t(x, dww_ref[...], pad_ref, H=H, W=W)
    h = _compact_w(acc, H=H, W=W) + dwb_col
    y = _ln_cols(h, lnw_col, lnb_col).astype(jnp.bfloat16)
    h1 = jnp.dot(w1t_ref[...], y, preferred_element_type=jnp.float32) + b1_col
    h1 = jax.nn.gelu(h1, approximate=True)
    z = jnp.dot(w2t_ref[...], h1.astype(jnp.bfloat16),
                preferred_element_type=jnp.float32) + b2_col
    return x + g_col * z


def _split4(o, dlnw_ref, dlnb_ref, o_refs, t_ref, *, H, W):
    C = o.shape[0]
    y = _ln_cols(o, _col(dlnw_ref[...]), _col(dlnb_ref[...]))
    yt = jnp.transpose(y)
    t_ref[...] = yt.reshape(H, W, C)
    M1 = (H // 2) * (W // 2)
    for k, (dh, dw) in enumerate(((0, 0), (0, 1), (1, 0), (1, 1))):
        o_refs[k][...] = (t_ref[dh::2, dw::2, :].reshape(M1, C)
                          .astype(o_refs[k].dtype))


def _k1(xp_ref, sw_ref, sb_ref, slnw_ref, slnb_ref,
        dww_ref, dwb_ref, lnw_ref, lnb_ref, w1t_ref, b1_ref, w2t_ref, b2_ref,
        g_ref, dlnw_ref, dlnb_ref, o0_ref, o1_ref, o2_ref, o3_ref, pad_ref,
        t_ref, *, H, W):
    s = jnp.dot(sw_ref[...], xp_ref[...],
                preferred_element_type=jnp.float32) + _col(sb_ref[...])
    s = _ln_cols(s, _col(slnw_ref[...]), _col(slnb_ref[...]))
    o = _block_t(s, dww_ref, _col(dwb_ref[...]), _col(lnw_ref[...]),
                 _col(lnb_ref[...]), w1t_ref, _col(b1_ref[...]), w2t_ref,
                 _col(b2_ref[...]), _col(g_ref[...]), pad_ref, H=H, W=W)
    _split4(o, dlnw_ref, dlnb_ref, (o0_ref, o1_ref, o2_ref, o3_ref), t_ref,
            H=H, W=W)


def _k23(p0_ref, p1_ref, p2_ref, p3_ref, dsw_ref, dsb_ref,
         dww_ref, dwb_ref, lnw_ref, lnb_ref, w1t_ref, b1_ref, w2t_ref, b2_ref,
         g_ref, dlnw_ref, dlnb_ref, *refs, H, W, last):
    acc = jnp.dot(p0_ref[...], dsw_ref[0], preferred_element_type=jnp.float32)
    acc = acc + jnp.dot(p1_ref[...], dsw_ref[1],
                        preferred_element_type=jnp.float32)
    acc = acc + jnp.dot(p2_ref[...], dsw_ref[2],
                        preferred_element_type=jnp.float32)
    acc = acc + jnp.dot(p3_ref[...], dsw_ref[3],
                        preferred_element_type=jnp.float32)
    x = jnp.transpose(acc + dsb_ref[...])
    pad_ref = refs[-1] if last else refs[-2]
    o = _block_t(x, dww_ref, _col(dwb_ref[...]), _col(lnw_ref[...]),
                 _col(lnb_ref[...]), w1t_ref, _col(b1_ref[...]), w2t_ref,
                 _col(b2_ref[...]), _col(g_ref[...]), pad_ref, H=H, W=W)
    if last:
        refs[0][...] = o
    else:
        _split4(o, dlnw_ref, dlnb_ref, refs[0:4], refs[-1], H=H, W=W)


def _full_spec(shape):
    n = len(shape)
    return pl.BlockSpec(shape, lambda i: (0,) * n)


def _img_spec(a, b):
    return pl.BlockSpec((None, a, b), lambda n: (n, 0, 0))


def _cp():
    return pltpu.CompilerParams(dimension_semantics=("parallel",),
                                vmem_limit_bytes=_VMEM_LIMIT)


def kernel(images, stem_conv_w, stem_conv_b, stem_ln_w, stem_ln_b, blk0_dw_w, blk0_dw_b, blk0_ln_w, blk0_ln_b, blk0_w1, blk0_b1, blk0_w2, blk0_b2, blk0_gamma, ds1_ln_w, ds1_ln_b, ds1_conv_w, ds1_conv_b, blk1_dw_w, blk1_dw_b, blk1_ln_w, blk1_ln_b, blk1_w1, blk1_b1, blk1_w2, blk1_b2, blk1_gamma, ds2_ln_w, ds2_ln_b, ds2_conv_w, ds2_conv_b, blk2_dw_w, blk2_dw_b, blk2_ln_w, blk2_ln_b, blk2_w1, blk2_b1, blk2_w2, blk2_b2, blk2_gamma, ds3_ln_w, ds3_ln_b, ds3_conv_w, ds3_conv_b, blk3_dw_w, blk3_dw_b, blk3_ln_w, blk3_ln_b, blk3_w1, blk3_b1, blk3_w2, blk3_b2, blk3_gamma):
    N, Cin, Him, Wim = images.shape
    ps = 4
    H0, W0 = Him // ps, Wim // ps
    M0 = H0 * W0
    C0, C1, C2 = blk0_dw_w.shape[1], blk1_dw_w.shape[1], blk2_dw_w.shape[1]
    K0 = Cin * ps * ps

    xp = images.reshape(N, Cin, H0, ps, W0, ps)
    xp = xp.transpose(0, 1, 3, 5, 2, 4).reshape(N, K0, M0)
    xp = xp.astype(jnp.bfloat16)
    swt = stem_conv_w.reshape(ps, ps, Cin, C0).transpose(3, 2, 0, 1)
    swt = swt.reshape(C0, K0).astype(jnp.bfloat16)

    H1, W1 = H0 // 2, W0 // 2
    M1 = H1 * W1
    H2, W2 = H1 // 2, W1 // 2
    M2 = H2 * W2

    Wp0 = W0 + 6
    k1 = functools.partial(_k1, H=H0, W=W0)
    pos_shape1 = jax.ShapeDtypeStruct((N, M1, C0), jnp.bfloat16)
    p = pl.pallas_call(
        k1,
        out_shape=[pos_shape1] * 4,
        grid=(N,),
        in_specs=[
            _img_spec(K0, M0),
            _full_spec((C0, K0)),
            _full_spec((1, C0)), _full_spec((1, C0)), _full_spec((1, C0)),
            _full_spec((C0, 49)), _full_spec((1, C0)),
            _full_spec((1, C0)), _full_spec((1, C0)),
            _full_spec((4 * C0, C0)), _full_spec((1, 4 * C0)),
            _full_spec((C0, 4 * C0)), _full_spec((1, C0)),
            _full_spec((1, C0)),
            _full_spec((1, C0)), _full_spec((1, C0)),
        ],
        out_specs=[_img_spec(M1, C0)] * 4,
        scratch_shapes=[pltpu.VMEM((C0, (H0 + 6) * Wp0 + 8), jnp.bfloat16),
                        pltpu.VMEM((H0, W0, C0), jnp.float32)],
        compiler_params=_cp(),
    )(xp, swt, _r2(stem_conv_b), _r2(stem_ln_w), _r2(stem_ln_b),
      blk0_dw_w.T, _r2(blk0_dw_b), _r2(blk0_ln_w), _r2(blk0_ln_b),
      blk0_w1.T.astype(jnp.bfloat16), _r2(blk0_b1),
      blk0_w2.T.astype(jnp.bfloat16), _r2(blk0_b2), _r2(blk0_gamma),
      _r2(ds1_ln_w), _r2(ds1_ln_b))

    Wp1 = W1 + 6
    k2 = functools.partial(_k23, H=H1, W=W1, last=False)
    pos_shape2 = jax.ShapeDtypeStruct((N, M2, C1), jnp.bfloat16)
    p = pl.pallas_call(
        k2,
        out_shape=[pos_shape2] * 4,
        grid=(N,),
        in_specs=[
            _img_spec(M1, C0), _img_spec(M1, C0),
            _img_spec(M1, C0), _img_spec(M1, C0),
            _full_spec((4, C0, C1)), _full_spec((1, C1)),
            _full_spec((C1, 49)), _full_spec((1, C1)),
            _full_spec((1, C1)), _full_spec((1, C1)),
            _full_spec((4 * C1, C1)), _full_spec((1, 4 * C1)),
            _full_spec((C1, 4 * C1)), _full_spec((1, C1)),
            _full_spec((1, C1)),
            _full_spec((1, C1)), _full_spec((1, C1)),
        ],
        out_specs=[_img_spec(M2, C1)] * 4,
        scratch_shapes=[pltpu.VMEM((C1, (H1 + 6) * Wp1 + 8), jnp.bfloat16),
                        pltpu.VMEM((H1, W1, C1), jnp.float32)],
        compiler_params=_cp(),
    )(*p, ds1_conv_w, _r2(ds1_conv_b),
      blk1_dw_w.T, _r2(blk1_dw_b), _r2(blk1_ln_w), _r2(blk1_ln_b),
      blk1_w1.T.astype(jnp.bfloat16), _r2(blk1_b1),
      blk1_w2.T.astype(jnp.bfloat16), _r2(blk1_b2), _r2(blk1_gamma),
      _r2(ds2_ln_w), _r2(ds2_ln_b))

    Wp2 = W2 + 6
    k3 = functools.partial(_k23, H=H2, W=W2, last=True)
    out = pl.pallas_call(
        k3,
        out_shape=jax.ShapeDtypeStruct((N, C2, M2), jnp.float32),
        grid=(N,),
        in_specs=[
            _img_spec(M2, C1), _img_spec(M2, C1),
            _img_spec(M2, C1), _img_spec(M2, C1),
            _full_spec((4, C1, C2)), _full_spec((1, C2)),
            _full_spec((C2, 49)), _full_spec((1, C2)),
            _full_spec((1, C2)), _full_spec((1, C2)),
            _full_spec((4 * C2, C2)), _full_spec((1, 4 * C2)),
            _full_spec((C2, 4 * C2)), _full_spec((1, C2)),
            _full_spec((1, C2)),
            _full_spec((1, C2)), _full_spec((1, C2)),
        ],
        out_specs=_img_spec(C2, M2),
        scratch_shapes=[pltpu.VMEM((C2, (H2 + 6) * Wp2 + 8), jnp.bfloat16)],
        compiler_params=_cp(),
    )(*p, ds2_conv_w, _r2(ds2_conv_b),
      blk2_dw_w.T, _r2(blk2_dw_b), _r2(blk2_ln_w), _r2(blk2_ln_b),
      blk2_w1.T.astype(jnp.bfloat16), _r2(blk2_b1),
      blk2_w2.T.astype(jnp.bfloat16), _r2(blk2_b2), _r2(blk2_gamma),
      _r2(ds3_ln_w), _r2(ds3_ln_b))

    return out.reshape(N, C2, H2, W2)
